# Initial kernel scaffold; baseline (speedup 1.0000x reference)
import jax
import jax.numpy as jnp
from jax.experimental import pallas as pl


def kernel(X, Y):
    raise NotImplementedError("write your pallas kernel here")


# speedup vs baseline: 4.5972x; 4.5972x over previous
import jax
import jax.numpy as jnp
from jax.experimental import pallas as pl
from jax.experimental.pallas import tpu as pltpu

N = 4096
BI = 512
GRID = N // BI


def _chamfer_body(xk0, xk1, xk2, y0, y1, y2, out_ref, colmin, rowacc):
    g = pl.program_id(0)

    d0 = xk0[...] - y0[...]
    d1 = xk1[...] - y1[...]
    d2c = xk2[...] - y2[...]
    d2 = d0 * d0 + d1 * d1 + d2c * d2c

    rmin = jnp.min(d2, axis=1)
    cmin = jnp.min(d2, axis=0, keepdims=True)

    @pl.when(g == 0)
    def _init():
        colmin[...] = cmin
        rowacc[0] = jnp.sum(jnp.sqrt(rmin))

    @pl.when(g > 0)
    def _acc():
        colmin[...] = jnp.minimum(colmin[...], cmin)
        rowacc[0] = rowacc[0] + jnp.sum(jnp.sqrt(rmin))

    @pl.when(g == GRID - 1)
    def _fin():
        loss2 = jnp.sum(jnp.sqrt(colmin[...]))
        out_ref[...] = jnp.full((1, 1), (rowacc[0] + loss2) * (1.0 / N),
                                dtype=jnp.float32)


def kernel(X, Y):
    Xf = X[0]
    Yf = Y[0]
    xs = [Xf[:, k].reshape(N, 1) for k in range(3)]
    ys = [Yf[:, k].reshape(1, N) for k in range(3)]

    out = pl.pallas_call(
        _chamfer_body,
        grid=(GRID,),
        in_specs=[
            pl.BlockSpec((BI, 1), lambda g: (g, 0)),
            pl.BlockSpec((BI, 1), lambda g: (g, 0)),
            pl.BlockSpec((BI, 1), lambda g: (g, 0)),
            pl.BlockSpec((1, N), lambda g: (0, 0)),
            pl.BlockSpec((1, N), lambda g: (0, 0)),
            pl.BlockSpec((1, N), lambda g: (0, 0)),
        ],
        out_specs=pl.BlockSpec((1, 1), lambda g: (0, 0)),
        out_shape=jax.ShapeDtypeStruct((1, 1), jnp.float32),
        scratch_shapes=[
            pltpu.VMEM((1, N), jnp.float32),
            pltpu.SMEM((1,), jnp.float32),
        ],
    )(*xs, *ys)
    return out[0, 0]
